# Initial kernel scaffold; baseline (speedup 1.0000x reference)
import jax
import jax.numpy as jnp
from jax.experimental import pallas as pl


def kernel(mem, idx, val, sample_idx):
    raise NotImplementedError("write your pallas kernel here")


# speedup vs baseline: 4.5998x; 4.5998x over previous
import functools

import jax
import jax.numpy as jnp
from jax import lax
from jax.experimental import pallas as pl
from jax.experimental.pallas import tpu as pltpu
from jax.experimental.pallas import tpu_sc as plsc

_CAP = 16384
_WB = 4096
_SB = 1024
_D = 3 * 32 * 32
_PAD_ROWS = 8
_NW = 32
_BPW = _SB // _NW


def _match_body(sample_ref, idx_ref, vidx_ref, dest_ref):
    i = pl.program_id(0)
    s = sample_ref[0]
    w = idx_ref[...]
    eq = w == s
    jio = lax.broadcasted_iota(jnp.int32, (_WB, 128), 0)
    lastj = jnp.max(jnp.where(eq, jio, -1), axis=0, keepdims=True)
    ow = lastj >= 0
    lanes = lax.broadcasted_iota(jnp.int32, (1, 128), 1)
    row = i * 128 + lanes
    vidx_ref[0] = jnp.where(ow, lastj, 0)
    dest_ref[0] = jnp.where(ow, row, _SB)


def _resolve_writes(sample3, idx2):
    return pl.pallas_call(
        _match_body,
        grid=(8,),
        in_specs=[
            pl.BlockSpec((1, 1, 128), lambda i: (i, 0, 0)),
            pl.BlockSpec((_WB, 1), lambda i: (0, 0)),
        ],
        out_specs=[
            pl.BlockSpec((1, 1, 128), lambda i: (i, 0, 0)),
            pl.BlockSpec((1, 1, 128), lambda i: (i, 0, 0)),
        ],
        out_shape=[
            jax.ShapeDtypeStruct((8, 1, 128), jnp.int32),
            jax.ShapeDtypeStruct((8, 1, 128), jnp.int32),
        ],
    )(sample3, idx2)


def _sc_gather(mem2, val2, sample, vidx, dest):
    @functools.partial(
        pl.kernel,
        mesh=plsc.VectorSubcoreMesh(core_axis_name="c", subcore_axis_name="s"),
        out_type=jax.ShapeDtypeStruct((_SB + _PAD_ROWS, _D), jnp.float32),
        scratch_types=[
            pltpu.VMEM((_BPW,), jnp.int32),
            pltpu.VMEM((_BPW,), jnp.int32),
            pltpu.VMEM((_BPW,), jnp.int32),
            pltpu.VMEM((_BPW, _D), jnp.float32),
            pltpu.SemaphoreType.DMA,
        ],
    )
    def k(mem_hbm, val_hbm, samp_hbm, vidx_hbm, dest_hbm, out_hbm,
          sidx_v, vidx_v, didx_v, buf_v, sem):
        wid = lax.axis_index("s") * 2 + lax.axis_index("c")
        base = wid * _BPW
        pltpu.sync_copy(samp_hbm.at[pl.ds(base, _BPW)], sidx_v)
        pltpu.async_copy(mem_hbm.at[sidx_v], buf_v, sem).wait()
        pltpu.sync_copy(buf_v, out_hbm.at[pl.ds(base, _BPW)])
        pltpu.sync_copy(vidx_hbm.at[pl.ds(base, _BPW)], vidx_v)
        pltpu.sync_copy(dest_hbm.at[pl.ds(base, _BPW)], didx_v)
        pltpu.async_copy(val_hbm.at[vidx_v], buf_v, sem).wait()
        pltpu.async_copy(buf_v, out_hbm.at[didx_v], sem).wait()

    return k(mem2, val2, sample, vidx, dest)


def kernel(mem, idx, val, sample_idx):
    mem2 = mem.reshape(_CAP, _D)
    val2 = val.reshape(_WB, _D)
    sample3 = sample_idx.reshape(8, 1, 128)
    idx2 = idx.reshape(_WB, 1)
    vidx, dest = _resolve_writes(sample3, idx2)
    outp = _sc_gather(mem2, val2, sample_idx,
                      vidx.reshape(_SB), dest.reshape(_SB))
    return outp[:_SB].reshape(_SB, 3, 32, 32)
